# baseline (speedup 1.0000x reference)
import functools

import jax
import jax.numpy as jnp
from jax import lax
from jax.experimental import pallas as pl
from jax.experimental.pallas import tpu as pltpu
from jax.experimental.pallas import tpu_sc as plsc

_N_ELEMENTS = 1000000
_D = 16
_B = 16384

_NC = 2
_NS = 16
_NW = _NC * _NS
_BPW = _B // _NW
_G = _BPW // 16

_PACK = 128 // _D
_NROWS = _N_ELEMENTS // _PACK
_ROWS128 = _B * _D // 128
_OPW = _ROWS128 // _NW


@functools.partial(
    pl.kernel,
    mesh=plsc.VectorSubcoreMesh(core_axis_name="c", subcore_axis_name="s"),
    out_type=[
        jax.ShapeDtypeStruct((_ROWS128, 128), jnp.float32),
        jax.ShapeDtypeStruct((_ROWS128, 128), jnp.float32),
    ],
    scratch_types=[
        pltpu.VMEM((_BPW,), jnp.int32),
        pltpu.VMEM((_BPW,), jnp.int32),
        pltpu.VMEM((_BPW, 128), jnp.float32),
        pltpu.VMEM((_OPW, 128), jnp.float32),
        pltpu.VMEM((_OPW, 128), jnp.float32),
        pltpu.SemaphoreType.DMA,
    ],
)
def _sc_gather(idx_hbm, mean_hbm, rawvar_hbm, out_m, out_v,
               idx_v, row_idx_v, rows_v, outbuf_m, outbuf_v, sem):
    wid = lax.axis_index("s") * _NC + lax.axis_index("c")
    base = wid * _BPW
    pltpu.sync_copy(idx_hbm.at[pl.ds(base, _BPW)], idx_v)

    def rbody(i, carry):
        row_idx_v[pl.ds(i * 16, 16)] = idx_v[pl.ds(i * 16, 16)] >> 3
        return carry

    lax.fori_loop(0, _G, rbody, 0)

    def extract(outbuf):
        def gbody(g, carry):
            sub = (idx_v[pl.ds(g * 16, 16)] & 7) * _D
            for l in range(16):
                s = sub[l]
                outbuf[g * 2 + (l >> 3), pl.ds((l & 7) * _D, _D)] = (
                    rows_v[g * 16 + l, pl.ds(s, _D)])
            return carry

        lax.fori_loop(0, _G, gbody, 0)

    pltpu.async_copy(mean_hbm.at[row_idx_v], rows_v, sem).wait()
    extract(outbuf_m)
    cp_v = pltpu.async_copy(rawvar_hbm.at[row_idx_v], rows_v, sem)
    pltpu.sync_copy(outbuf_m, out_m.at[pl.ds(wid * _OPW, _OPW)])
    cp_v.wait()
    extract(outbuf_v)
    pltpu.sync_copy(outbuf_v, out_v.at[pl.ds(wid * _OPW, _OPW)])


def _softplus_stack_body(m_ref, v_ref, o_ref):
    o_ref[0] = m_ref[:]
    x = v_ref[:]
    o_ref[1] = jnp.maximum(x, 0.0) + jnp.log1p(jnp.exp(-jnp.abs(x)))


def _softplus_stack(m2, v2):
    return pl.pallas_call(
        _softplus_stack_body,
        out_shape=jax.ShapeDtypeStruct((2, _ROWS128, 128), jnp.float32),
    )(m2, v2)


def kernel(indices, variational_mean, raw_variational_variance):
    idx = indices.astype(jnp.int32)
    mean2 = variational_mean.reshape(_NROWS, 128)
    var2 = raw_variational_variance.reshape(_NROWS, 128)
    ms, vs_raw = _sc_gather(idx, mean2, var2)
    out = _softplus_stack(ms, vs_raw)
    return out.reshape(2, _B, _D)


# speedup vs baseline: 1.4885x; 1.4885x over previous
import functools

import jax
import jax.numpy as jnp
from jax import lax
from jax.experimental import pallas as pl
from jax.experimental.pallas import tpu as pltpu
from jax.experimental.pallas import tpu_sc as plsc

_N_ELEMENTS = 1000000
_D = 16
_B = 16384

_NC = 2
_NS = 16
_NW = _NC * _NS
_BPW = _B // _NW
_HW = _BPW // 2

_ROWS128 = _B * _D // 128
_OPW = _ROWS128 // _NW


@functools.partial(
    pl.kernel,
    mesh=plsc.VectorSubcoreMesh(core_axis_name="c", subcore_axis_name="s"),
    out_type=[
        jax.ShapeDtypeStruct((_ROWS128, 128), jnp.float32),
        jax.ShapeDtypeStruct((_ROWS128, 128), jnp.float32),
    ],
    scratch_types=[
        pltpu.VMEM((_BPW,), jnp.int32),
        pltpu.VMEM((_HW, _D), jnp.float32),
        pltpu.VMEM((_HW, _D), jnp.float32),
        pltpu.VMEM((_OPW, 128), jnp.float32),
        pltpu.VMEM((_OPW, 128), jnp.float32),
        pltpu.SemaphoreType.DMA,
        pltpu.SemaphoreType.DMA,
    ],
)
def _sc_gather(idx_hbm, mean_hbm, rawvar_hbm, out_m, out_v,
               idx_v, buf_m, buf_v, outc_m, outc_v, sem_m, sem_v):
    wid = lax.axis_index("s") * _NC + lax.axis_index("c")
    base = wid * _BPW
    pltpu.sync_copy(idx_hbm.at[pl.ds(base, _BPW)], idx_v)

    def fetch(tbl, buf, sem, wave):
        def gbody(g, carry):
            vec = idx_v[pl.ds(wave * _HW + g * 16, 16)]
            for l in range(16):
                pltpu.async_copy(
                    tbl.at[pl.ds(vec[l], 1), :],
                    buf.at[pl.ds(g * 16 + l, 1), :], sem)
            return carry

        lax.fori_loop(0, _HW // 16, gbody, 0)

    def drain(buf, sem):
        pltpu.make_async_copy(mean_hbm.at[pl.ds(0, _HW), :], buf, sem).wait()

    def repack(buf, outc, wave):
        def gbody(g, carry):
            for l in range(16):
                outc[wave * (_HW >> 3) + g * 2 + (l >> 3),
                     pl.ds((l & 7) * _D, _D)] = buf[g * 16 + l, :]
            return carry

        lax.fori_loop(0, _HW // 16, gbody, 0)

    for wave in range(2):
        fetch(mean_hbm, buf_m, sem_m, wave)
        fetch(rawvar_hbm, buf_v, sem_v, wave)
        drain(buf_m, sem_m)
        repack(buf_m, outc_m, wave)
        drain(buf_v, sem_v)
        repack(buf_v, outc_v, wave)

    pltpu.sync_copy(outc_m, out_m.at[pl.ds(wid * _OPW, _OPW)])
    pltpu.sync_copy(outc_v, out_v.at[pl.ds(wid * _OPW, _OPW)])


def _softplus_stack_body(m_ref, v_ref, o_ref):
    o_ref[0] = m_ref[:]
    x = v_ref[:]
    o_ref[1] = jnp.maximum(x, 0.0) + jnp.log1p(jnp.exp(-jnp.abs(x)))


def _softplus_stack(m2, v2):
    return pl.pallas_call(
        _softplus_stack_body,
        out_shape=jax.ShapeDtypeStruct((2, _ROWS128, 128), jnp.float32),
    )(m2, v2)


def kernel(indices, variational_mean, raw_variational_variance):
    idx = indices.astype(jnp.int32)
    ms, vs_raw = _sc_gather(idx, variational_mean, raw_variational_variance)
    out = _softplus_stack(ms, vs_raw)
    return out.reshape(2, _B, _D)
